# Initial kernel scaffold; baseline (speedup 1.0000x reference)
import jax
import jax.numpy as jnp
from jax.experimental import pallas as pl


def kernel(item_embeddings_list, W):
    raise NotImplementedError("write your pallas kernel here")


# speedup vs baseline: 1.2258x; 1.2258x over previous
import functools
import jax
import jax.numpy as jnp
from jax import lax
from jax.experimental import pallas as pl
from jax.experimental.pallas import tpu as pltpu
from jax.experimental.pallas import tpu_sc as plsc

B, L, D = 4096, 200, 64
NC, NS = 2, 16
NW = NC * NS
UPW = B // NW
LANES = 16
NCH = D // LANES
OB_USERS = 16
SPAD = 208
NG = SPAD // LANES


def _compute_user(xb, sc, ob, wv, ps, umod):
    w0 = wv[pl.ds(0, LANES)]
    w1 = wv[pl.ds(16, LANES)]
    w2 = wv[pl.ds(32, LANES)]
    w3 = wv[pl.ds(48, LANES)]
    lanes16 = jnp.arange(LANES, dtype=jnp.int32) * LANES

    @plsc.parallel_loop(0, SPAD, unroll=4)
    def _(j):
        off = j * D
        p = (xb[pl.ds(off, LANES)] * w0
             + xb[pl.ds(off + 16, LANES)] * w1
             + xb[pl.ds(off + 32, LANES)] * w2
             + xb[pl.ds(off + 48, LANES)] * w3)
        ps[pl.ds(j * LANES, LANES)] = p

    neg_big = jnp.full((LANES,), -1e30, jnp.float32)

    @plsc.parallel_loop(0, NG - 1, unroll=2, carry=neg_big)
    def m01(g, mcar):
        pbase = g * (LANES * LANES)
        cols = [plsc.load_gather(ps, [lanes16 + (pbase + k)])
                for k in range(LANES)]
        while len(cols) > 1:
            cols = [a + b for a, b in zip(cols[::2], cols[1::2])]
        sc[pl.ds(g * LANES, LANES)] = cols[0]
        return jnp.maximum(mcar, cols[0])

    pbase = (NG - 1) * (LANES * LANES)
    cols = [plsc.load_gather(ps, [lanes16 + (pbase + k)])
            for k in range(LANES)]
    while len(cols) > 1:
        cols = [a + b for a, b in zip(cols[::2], cols[1::2])]
    padmask = jnp.arange(LANES, dtype=jnp.int32) < 8
    stail = jnp.where(padmask, cols[0], neg_big)
    sc[pl.ds(L - 8, LANES)] = stail

    m = jnp.maximum(m01, stail)
    mm = jnp.max(m)
    zv = jnp.zeros((LANES,), dtype=jnp.float32)
    for g in range(NG):
        e = jnp.exp(sc[pl.ds(g * LANES, LANES)] - mm)
        sc[pl.ds(g * LANES, LANES)] = e
        zv = zv + e
    rz = jnp.full((LANES,), 1.0, jnp.float32) / (
        jnp.zeros((LANES,), jnp.float32) + jnp.sum(zv))

    def body_b(i, carry):
        a0, a1, a2, a3 = carry
        ev = sc[pl.ds(i * LANES, LANES)]
        rbase = i * (LANES * D)
        for k in range(LANES):
            e = ev[k]
            off = rbase + k * D
            a0 = a0 + e * xb[pl.ds(off, LANES)]
            a1 = a1 + e * xb[pl.ds(off + 16, LANES)]
            a2 = a2 + e * xb[pl.ds(off + 32, LANES)]
            a3 = a3 + e * xb[pl.ds(off + 48, LANES)]
        return a0, a1, a2, a3

    z16 = jnp.zeros((LANES,), dtype=jnp.float32)
    o0, o1, o2, o3 = lax.fori_loop(0, NG, body_b, (z16, z16, z16, z16),
                                   unroll=False)
    obase = umod * D
    ob[pl.ds(obase, LANES)] = o0 * rz
    ob[pl.ds(obase + 16, LANES)] = o1 * rz
    ob[pl.ds(obase + 32, LANES)] = o2 * rz
    ob[pl.ds(obase + 48, LANES)] = o3 * rz


def _sc_kernel(x_hbm, w_hbm, out_hbm, xba, xbb, wv, sc, ob, ps, sema, semb):
    wid = lax.axis_index("s") * NC + lax.axis_index("c")
    base = wid * UPW
    pltpu.sync_copy(w_hbm, wv)
    z16 = jnp.zeros((LANES,), jnp.float32)
    for buf in (xba, xbb):
        for off in range(L * D, SPAD * D, LANES):
            buf[pl.ds(off, LANES)] = z16

    pltpu.async_copy(x_hbm.at[base], xba.at[pl.ds(0, L * D)], sema)

    def pair_body(p, _):
        ua = base + p * 2
        pltpu.async_copy(x_hbm.at[ua + 1], xbb.at[pl.ds(0, L * D)], semb)
        pltpu.make_async_copy(x_hbm.at[ua], xba.at[pl.ds(0, L * D)],
                              sema).wait()
        _compute_user(xba, sc, ob, wv, ps, (p * 2) % OB_USERS)
        nxt = jnp.minimum(ua + 2, base + UPW - 1)
        pltpu.async_copy(x_hbm.at[nxt], xba.at[pl.ds(0, L * D)], sema)
        pltpu.make_async_copy(x_hbm.at[ua + 1], xbb.at[pl.ds(0, L * D)],
                              semb).wait()
        _compute_user(xbb, sc, ob, wv, ps, (p * 2 + 1) % OB_USERS)

        @pl.when((p % (OB_USERS // 2)) == (OB_USERS // 2 - 1))
        def _():
            blk = base + (p // (OB_USERS // 2)) * OB_USERS
            pltpu.sync_copy(ob, out_hbm.at[pl.ds(blk * D, OB_USERS * D)])

        return 0

    lax.fori_loop(0, UPW // 2, pair_body, 0, unroll=False)
    pltpu.make_async_copy(x_hbm.at[base + UPW - 1], xba.at[pl.ds(0, L * D)],
                          sema).wait()


@jax.jit
def _attention_pool(x2d, w):
    mesh = plsc.VectorSubcoreMesh(core_axis_name="c", subcore_axis_name="s",
                                  num_cores=NC)
    f = functools.partial(
        pl.kernel,
        out_type=jax.ShapeDtypeStruct((B * D,), jnp.float32),
        mesh=mesh,
        scratch_types=[
            pltpu.VMEM((SPAD * D,), jnp.float32),
            pltpu.VMEM((SPAD * D,), jnp.float32),
            pltpu.VMEM((D,), jnp.float32),
            pltpu.VMEM((SPAD,), jnp.float32),
            pltpu.VMEM((OB_USERS * D,), jnp.float32),
            pltpu.VMEM((SPAD * LANES,), jnp.float32),
            pltpu.SemaphoreType.DMA,
            pltpu.SemaphoreType.DMA,
        ],
        compiler_params=pltpu.CompilerParams(needs_layout_passes=False),
    )(_sc_kernel)
    return f(x2d, w)


def kernel(item_embeddings_list, W):
    x2d = item_embeddings_list.reshape(B, L * D)
    w = W.reshape(D)
    return _attention_pool(x2d, w).reshape(B, D)
